# baseline (speedup 1.0000x reference)
import functools

import jax
import jax.numpy as jnp
from jax import lax
from jax.experimental import pallas as pl
from jax.experimental.pallas import tpu as pltpu
from jax.experimental.pallas import tpu_sc as plsc

N = 10000
E = 320000
D = 128
NC = 2
NS = 16
NW = NC * NS
EPW = E // NW
K = 80
ITERS = EPW // K
NSTAGE = 5
IPS = ITERS // NSTAGE
NP = 10240
RPT = NP // NS


def _sc_gather_scatter(feat, ei):
    mesh = plsc.VectorSubcoreMesh(core_axis_name="c", subcore_axis_name="s")

    @functools.partial(
        pl.kernel,
        out_type=jax.ShapeDtypeStruct((NC, NP, D), jnp.float32),
        mesh=mesh,
        scratch_types=[
            pltpu.VMEM((IPS, K), jnp.int32),
            pltpu.VMEM((IPS, K), jnp.int32),
            pltpu.VMEM((IPS, K), jnp.int32),
            pltpu.VMEM((IPS, K), jnp.int32),
            pltpu.VMEM((K, D), jnp.float32),
            pltpu.VMEM((K, D), jnp.float32),
            pltpu.VMEM_SHARED((NP, D), jnp.float32),
            pltpu.SemaphoreType.DMA,
            pltpu.SemaphoreType.DMA,
            pltpu.SemaphoreType.DMA,
            pltpu.SemaphoreType.DMA,
            pltpu.SemaphoreType.DMA,
            pltpu.SemaphoreType.DMA,
        ],
    )
    def k(feat_hbm, ei_hbm, out_hbm, sixa, sixb, dixa, dixb, rows0, rows1,
          accum, g0, g1, s0, s1, ia, ib):
        c = lax.axis_index("c")
        s = lax.axis_index("s")
        wid = c * NS + s
        six = (sixa, sixb)
        dix = (dixa, dixb)
        isem = (ia, ib)

        pltpu.async_copy(ei_hbm.at[0, wid, 0], six[0], isem[0])
        pltpu.async_copy(ei_hbm.at[1, wid, 0], dix[0], isem[0])

        z = jnp.zeros((16,), jnp.float32)

        def zrow(i, _):
            for j in range(D // 16):
                rows0[i, pl.ds(j * 16, 16)] = z
            return 0

        lax.fori_loop(0, K, zrow, 0)
        r0 = s * RPT
        for j in range(RPT // K):
            pltpu.sync_copy(rows0, accum.at[pl.ds(r0 + j * K, K)])
        plsc.subcore_barrier()

        def gather(sx, ch, rows, sem):
            return pltpu.async_copy(feat_hbm.at[sx.at[ch]], rows, sem)

        def gather_wait(sx, ch, rows, sem):
            pltpu.make_async_copy(feat_hbm.at[sx.at[ch]], rows, sem).wait()

        def scat(dx, ch, rows, sem):
            return pltpu.async_copy(rows, accum.at[dx.at[ch]], sem, add=True)

        def scat_wait(dx, ch, rows, sem):
            pltpu.make_async_copy(rows, accum.at[dx.at[ch]], sem).wait()

        for sg in range(NSTAGE):
            p = sg % 2
            sx, dx = six[p], dix[p]
            pltpu.make_async_copy(ei_hbm.at[0, wid, sg], sx, isem[p]).wait()
            pltpu.make_async_copy(ei_hbm.at[1, wid, sg], dx, isem[p]).wait()
            if sg + 1 < NSTAGE:
                q = (sg + 1) % 2
                pltpu.async_copy(ei_hbm.at[0, wid, sg + 1], six[q], isem[q])
                pltpu.async_copy(ei_hbm.at[1, wid, sg + 1], dix[q], isem[q])

            gather(sx, 0, rows0, g0)
            gather(sx, 1, rows1, g1)

            def body(i, _):
                c0 = 2 * i
                gather_wait(sx, c0, rows0, g0)
                scat(dx, c0, rows0, s0)
                gather_wait(sx, c0 + 1, rows1, g1)
                scat(dx, c0 + 1, rows1, s1)
                scat_wait(dx, c0, rows0, s0)
                gather(sx, c0 + 2, rows0, g0)

                @pl.when(c0 + 3 < IPS)
                def _():
                    scat_wait(dx, c0 + 1, rows1, s1)
                    gather(sx, c0 + 3, rows1, g1)

                return 0

            lax.fori_loop(0, (IPS - 1) // 2, body, 0)
            scat_wait(dx, IPS - 2, rows1, s1)
            gather_wait(sx, IPS - 1, rows0, g0)
            scat(dx, IPS - 1, rows0, s0)
            scat_wait(dx, IPS - 1, rows0, s0)

        plsc.subcore_barrier()

        pltpu.sync_copy(accum.at[pl.ds(r0, RPT)], out_hbm.at[c, pl.ds(r0, RPT)])

    return k(feat, ei)


def _tc_final(partials, feat, w_neigh, w_self, b_neigh):
    bn = 1000

    def body(p_ref, f_ref, wn_ref, ws_ref, b_ref, o_ref):
        agg = p_ref[0] + p_ref[1]
        o_ref[...] = (
            jnp.dot(agg, wn_ref[...], preferred_element_type=jnp.float32)
            + jnp.dot(f_ref[...], ws_ref[...], preferred_element_type=jnp.float32)
            + b_ref[...]
        )

    return pl.pallas_call(
        body,
        grid=(N // bn,),
        in_specs=[
            pl.BlockSpec((NC, bn, D), lambda i: (0, i, 0)),
            pl.BlockSpec((bn, D), lambda i: (i, 0)),
            pl.BlockSpec((D, D), lambda i: (0, 0)),
            pl.BlockSpec((D, D), lambda i: (0, 0)),
            pl.BlockSpec((1, D), lambda i: (0, 0)),
        ],
        out_specs=pl.BlockSpec((bn, D), lambda i: (i, 0)),
        out_shape=jax.ShapeDtypeStruct((N, D), jnp.float32),
    )(partials, feat, w_neigh, w_self, b_neigh.reshape(1, D))


def kernel(feat, edge_index, W_neigh, b_neigh, W_self):
    ei = edge_index.reshape(2, NW, NSTAGE, IPS, K)
    partials = _sc_gather_scatter(feat, ei)
    return _tc_final(partials, feat, W_neigh, W_self, b_neigh)


# speedup vs baseline: 1.2119x; 1.2119x over previous
import functools

import jax
import jax.numpy as jnp
from jax import lax
from jax.experimental import pallas as pl
from jax.experimental.pallas import tpu as pltpu
from jax.experimental.pallas import tpu_sc as plsc

N = 10000
E = 320000
D = 128
NC = 2
NS = 16
NW = NC * NS
EPW = E // NW
K = 80
ITERS = EPW // K
NSTAGE = 5
IPS = ITERS // NSTAGE
NP = 10240
RPT = NP // NS


def _sc_gather_scatter(feat, ei):
    mesh = plsc.VectorSubcoreMesh(core_axis_name="c", subcore_axis_name="s")

    @functools.partial(
        pl.kernel,
        out_type=jax.ShapeDtypeStruct((NC, NP, D), jnp.float32),
        mesh=mesh,
        scratch_types=[
            pltpu.VMEM((IPS, K), jnp.int32),
            pltpu.VMEM((IPS, K), jnp.int32),
            pltpu.VMEM((IPS, K), jnp.int32),
            pltpu.VMEM((IPS, K), jnp.int32),
            pltpu.VMEM((K, D), jnp.float32),
            pltpu.VMEM((K, D), jnp.float32),
            pltpu.VMEM_SHARED((NP, D), jnp.float32),
            pltpu.SemaphoreType.DMA,
            pltpu.SemaphoreType.DMA,
            pltpu.SemaphoreType.DMA,
            pltpu.SemaphoreType.DMA,
        ],
    )
    def k(feat_hbm, ei_hbm, out_hbm, sixa, sixb, dixa, dixb, rows0, rows1,
          accum, g0, g1, ia, ib):
        c = lax.axis_index("c")
        s = lax.axis_index("s")
        wid = c * NS + s
        six = (sixa, sixb)
        dix = (dixa, dixb)
        isem = (ia, ib)

        pltpu.async_copy(ei_hbm.at[0, wid, 0], six[0], isem[0])
        pltpu.async_copy(ei_hbm.at[1, wid, 0], dix[0], isem[0])

        z = jnp.zeros((16,), jnp.float32)

        def zrow(i, _):
            for j in range(D // 16):
                rows0[i, pl.ds(j * 16, 16)] = z
            return 0

        lax.fori_loop(0, K, zrow, 0)
        r0 = s * RPT
        for j in range(RPT // K):
            pltpu.sync_copy(rows0, accum.at[pl.ds(r0 + j * K, K)])
        plsc.subcore_barrier()

        def gather(sx, ch, rows, sem):
            return pltpu.async_copy(feat_hbm.at[sx.at[ch]], rows, sem)

        def gather_wait(sx, ch, rows, sem):
            pltpu.make_async_copy(feat_hbm.at[sx.at[ch]], rows, sem).wait()

        for sg in range(NSTAGE):
            p = sg % 2
            sx, dx = six[p], dix[p]
            pltpu.make_async_copy(ei_hbm.at[0, wid, sg], sx, isem[p]).wait()
            pltpu.make_async_copy(ei_hbm.at[1, wid, sg], dx, isem[p]).wait()
            if sg + 1 < NSTAGE:
                q = (sg + 1) % 2
                pltpu.async_copy(ei_hbm.at[0, wid, sg + 1], six[q], isem[q])
                pltpu.async_copy(ei_hbm.at[1, wid, sg + 1], dix[q], isem[q])

            gather(sx, 0, rows0, g0)

            def body(i, _):
                c0 = 2 * i
                gather(sx, c0 + 1, rows1, g1)
                gather_wait(sx, c0, rows0, g0)
                pltpu.sync_copy(rows0, accum.at[dx.at[c0]], add=True)
                gather(sx, c0 + 2, rows0, g0)
                gather_wait(sx, c0 + 1, rows1, g1)
                pltpu.sync_copy(rows1, accum.at[dx.at[c0 + 1]], add=True)
                return 0

            lax.fori_loop(0, (IPS - 1) // 2, body, 0)
            gather_wait(sx, IPS - 1, rows0, g0)
            pltpu.sync_copy(rows0, accum.at[dx.at[IPS - 1]], add=True)

        plsc.subcore_barrier()

        pltpu.sync_copy(accum.at[pl.ds(r0, RPT)], out_hbm.at[c, pl.ds(r0, RPT)])

    return k(feat, ei)


def _tc_final(partials, feat, w_neigh, w_self, b_neigh):
    bn = 1000

    def body(p_ref, f_ref, wn_ref, ws_ref, b_ref, o_ref):
        agg = p_ref[0] + p_ref[1]
        o_ref[...] = (
            jnp.dot(agg, wn_ref[...], preferred_element_type=jnp.float32)
            + jnp.dot(f_ref[...], ws_ref[...], preferred_element_type=jnp.float32)
            + b_ref[...]
        )

    return pl.pallas_call(
        body,
        grid=(N // bn,),
        in_specs=[
            pl.BlockSpec((NC, bn, D), lambda i: (0, i, 0)),
            pl.BlockSpec((bn, D), lambda i: (i, 0)),
            pl.BlockSpec((D, D), lambda i: (0, 0)),
            pl.BlockSpec((D, D), lambda i: (0, 0)),
            pl.BlockSpec((1, D), lambda i: (0, 0)),
        ],
        out_specs=pl.BlockSpec((bn, D), lambda i: (i, 0)),
        out_shape=jax.ShapeDtypeStruct((N, D), jnp.float32),
    )(partials, feat, w_neigh, w_self, b_neigh.reshape(1, D))


def kernel(feat, edge_index, W_neigh, b_neigh, W_self):
    ei = edge_index.reshape(2, NW, NSTAGE, IPS, K)
    partials = _sc_gather_scatter(feat, ei)
    return _tc_final(partials, feat, W_neigh, W_self, b_neigh)


# speedup vs baseline: 1.3697x; 1.1302x over previous
import functools

import jax
import jax.numpy as jnp
from jax import lax
from jax.experimental import pallas as pl
from jax.experimental.pallas import tpu as pltpu
from jax.experimental.pallas import tpu_sc as plsc

N = 10000
E = 320000
D = 128
NC = 2
NS = 16
NW = NC * NS
EPW = E // NW
K = 80
ITERS = EPW // K
NSTAGE = 5
IPS = ITERS // NSTAGE
NP = 10240
RPT = NP // NS


def _sc_gather_scatter(feat, ei):
    mesh = plsc.VectorSubcoreMesh(core_axis_name="c", subcore_axis_name="s")

    @functools.partial(
        pl.kernel,
        out_type=jax.ShapeDtypeStruct((NC, NP, D), jnp.float32),
        mesh=mesh,
        scratch_types=[
            pltpu.VMEM((IPS, K), jnp.int32),
            pltpu.VMEM((IPS, K), jnp.int32),
            pltpu.VMEM((IPS, K), jnp.int32),
            pltpu.VMEM((IPS, K), jnp.int32),
            pltpu.VMEM((K, D), jnp.float32),
            pltpu.VMEM((K, D), jnp.float32),
            pltpu.VMEM((K, D), jnp.float32),
            pltpu.VMEM_SHARED((NP, D), jnp.float32),
            pltpu.SemaphoreType.DMA,
            pltpu.SemaphoreType.DMA,
            pltpu.SemaphoreType.DMA,
            pltpu.SemaphoreType.DMA,
            pltpu.SemaphoreType.DMA,
            pltpu.SemaphoreType.DMA,
            pltpu.SemaphoreType.DMA,
            pltpu.SemaphoreType.DMA,
            pltpu.SemaphoreType.DMA,
        ],
    )
    def k(feat_hbm, ei_hbm, out_hbm, sixa, sixb, dixa, dixb, rows0, rows1,
          rows2, accum, g0, g1, g2, s0, s1, s2, ia, ib, zsem):
        c = lax.axis_index("c")
        s = lax.axis_index("s")
        wid = c * NS + s
        six = (sixa, sixb)
        dix = (dixa, dixb)
        isem = (ia, ib)
        rows = (rows0, rows1, rows2)
        gsem = (g0, g1, g2)
        ssem = (s0, s1, s2)

        pltpu.async_copy(ei_hbm.at[0, wid, 0], six[0], isem[0])
        pltpu.async_copy(ei_hbm.at[1, wid, 0], dix[0], isem[0])

        z = jnp.zeros((16,), jnp.float32)

        def zrow(i, _):
            for j in range(D // 16):
                rows0[i, pl.ds(j * 16, 16)] = z
            return 0

        lax.fori_loop(0, K, zrow, 0)
        r0 = s * RPT
        for j in range(RPT // K):
            pltpu.async_copy(rows0, accum.at[pl.ds(r0 + j * K, K)], zsem)
        for j in range(RPT // K):
            pltpu.make_async_copy(rows0, accum.at[pl.ds(r0, K)], zsem).wait()
        plsc.subcore_barrier()

        def gat(sx, ch, b):
            return pltpu.async_copy(feat_hbm.at[sx.at[ch]], rows[b], gsem[b])

        def gat_w(sx, ch, b):
            pltpu.make_async_copy(feat_hbm.at[sx.at[ch]], rows[b], gsem[b]).wait()

        def sca(dx, ch, b):
            return pltpu.async_copy(rows[b], accum.at[dx.at[ch]], ssem[b],
                                    add=True)

        def sca_w(dx, ch, b):
            pltpu.make_async_copy(rows[b], accum.at[dx.at[ch]], ssem[b]).wait()

        for sg in range(NSTAGE):
            p = sg % 2
            sx, dx = six[p], dix[p]
            pltpu.make_async_copy(ei_hbm.at[0, wid, sg], sx, isem[p]).wait()
            pltpu.make_async_copy(ei_hbm.at[1, wid, sg], dx, isem[p]).wait()
            if sg + 1 < NSTAGE:
                q = (sg + 1) % 2
                pltpu.async_copy(ei_hbm.at[0, wid, sg + 1], six[q], isem[q])
                pltpu.async_copy(ei_hbm.at[1, wid, sg + 1], dix[q], isem[q])

            gat(sx, 0, 0)
            gat(sx, 1, 1)
            gat_w(sx, 0, 0)
            sca(dx, 0, 0)
            gat(sx, 2, 2)
            gat_w(sx, 1, 1)
            sca(dx, 1, 1)
            sca_w(dx, 0, 0)
            gat(sx, 3, 0)
            gat_w(sx, 2, 2)
            sca(dx, 2, 2)
            sca_w(dx, 1, 1)
            gat(sx, 4, 1)

            def body(i, _):
                c0 = 3 + 3 * i
                gat_w(sx, c0, 0)
                sca(dx, c0, 0)
                sca_w(dx, c0 - 1, 2)
                gat(sx, c0 + 2, 2)
                gat_w(sx, c0 + 1, 1)
                sca(dx, c0 + 1, 1)
                sca_w(dx, c0, 0)
                gat(sx, c0 + 3, 0)
                gat_w(sx, c0 + 2, 2)
                sca(dx, c0 + 2, 2)
                sca_w(dx, c0 + 1, 1)

                @pl.when(c0 + 4 < IPS)
                def _():
                    gat(sx, c0 + 4, 1)

                return 0

            lax.fori_loop(0, (IPS - 4) // 3, body, 0)
            gat_w(sx, IPS - 1, 0)
            sca(dx, IPS - 1, 0)
            sca_w(dx, IPS - 2, 2)
            sca_w(dx, IPS - 1, 0)

        plsc.subcore_barrier()

        pltpu.sync_copy(accum.at[pl.ds(r0, RPT)], out_hbm.at[c, pl.ds(r0, RPT)])

    return k(feat, ei)


def _tc_final(partials, feat, w_neigh, w_self, b_neigh):
    bn = 1000

    def body(p_ref, f_ref, wn_ref, ws_ref, b_ref, o_ref):
        agg = p_ref[0] + p_ref[1]
        o_ref[...] = (
            jnp.dot(agg, wn_ref[...], preferred_element_type=jnp.float32)
            + jnp.dot(f_ref[...], ws_ref[...], preferred_element_type=jnp.float32)
            + b_ref[...]
        )

    return pl.pallas_call(
        body,
        grid=(N // bn,),
        in_specs=[
            pl.BlockSpec((NC, bn, D), lambda i: (0, i, 0)),
            pl.BlockSpec((bn, D), lambda i: (i, 0)),
            pl.BlockSpec((D, D), lambda i: (0, 0)),
            pl.BlockSpec((D, D), lambda i: (0, 0)),
            pl.BlockSpec((1, D), lambda i: (0, 0)),
        ],
        out_specs=pl.BlockSpec((bn, D), lambda i: (i, 0)),
        out_shape=jax.ShapeDtypeStruct((N, D), jnp.float32),
    )(partials, feat, w_neigh, w_self, b_neigh.reshape(1, D))


def kernel(feat, edge_index, W_neigh, b_neigh, W_self):
    ei = edge_index.reshape(2, NW, NSTAGE, IPS, K)
    partials = _sc_gather_scatter(feat, ei)
    return _tc_final(partials, feat, W_neigh, W_self, b_neigh)
